# baseline (speedup 1.0000x reference)
import jax
import jax.numpy as jnp
import numpy as np
from jax.experimental import pallas as pl

B = 2
NB = 12000
C = 64
NKP = 2048
NS = 16
NZ, NY, NX = 5, 200, 176
VOXEL_SIZE = np.array([0.05, 0.05, 0.1], np.float32)
PC_RANGE = np.array([0.0, -40.0, -3.0, 70.4, 40.0, 1.0], np.float32)
DEFORM_R = 2.0
POOL_R = 1.2
DEC_R = 1.6


def _fps(xyz, npoint):
    n = xyz.shape[0]

    def body(i, st):
        idxs, dists, last = st
        d = jnp.sum((xyz - xyz[last]) ** 2, -1)
        dists = jnp.minimum(dists, d)
        nxt = jnp.argmax(dists).astype(jnp.int32)
        return idxs.at[i].set(nxt), dists, nxt

    idxs = jnp.zeros((npoint,), jnp.int32)
    dists = jnp.full((n,), 1e10, jnp.float32)
    idxs, _, _ = jax.lax.fori_loop(1, npoint, body, (idxs, dists, jnp.int32(0)))
    return idxs


def _ball_query(new_xyz, xyz, radius, nsample):
    n = xyz.shape[0]
    d2 = jnp.sum((new_xyz[:, None, :] - xyz[None, :, :]) ** 2, -1)
    within = d2 < radius * radius
    order = jnp.where(within, jnp.arange(n, dtype=jnp.int32)[None, :], n)
    neg_vals, _ = jax.lax.top_k(-order, nsample)
    idx = -neg_vals
    empty = ~jnp.any(within, axis=1)
    first = idx[:, :1]
    idx = jnp.where(idx == n, first, idx)
    idx = jnp.where(empty[:, None], 0, idx)
    return idx, empty


def _grouped_pointnet(xyz_src, feat_src, centers, idx, empty, W1, b1, W2, b2):
    rel = xyz_src[idx] - centers[:, None, :]
    gf = jnp.concatenate([rel, feat_src[idx]], -1)
    h = jax.nn.relu(gf @ W1 + b1)
    h = jax.nn.relu(h @ W2 + b2)
    out = jnp.max(h, axis=1)
    return jnp.where(empty[:, None], 0.0, out)


def _sa_block(feats, p):
    t = feats @ p["Wt"]
    q = feats @ p["Wp"]
    g = feats @ p["Wg"]
    attn = jax.nn.softmax(t @ q.T, axis=-1)
    out = attn @ g
    og = out.reshape(-1, 4, C // 4)
    z = jnp.einsum('ngi,gij->ngj', og, p["Wz"]).reshape(-1, C)
    mu = z.mean()
    var = z.var()
    zn = (z - mu) / jnp.sqrt(var + 1e-5) * p["gamma"] + p["beta"]
    return zn + feats


def kernel(voxel_features, params, voxel_coords):
    coords = voxel_coords
    vs = jnp.asarray(VOXEL_SIZE) * 8.0
    pcr_lo = jnp.asarray(PC_RANGE[:3])
    pcr_hi = jnp.asarray(PC_RANGE[3:])
    xyz = (coords[:, jnp.array([3, 2, 1])].astype(jnp.float32) + 0.5) * vs + pcr_lo
    xyz_b = xyz.reshape(B, NB, 3)
    feat_b = voxel_features.reshape(B, NB, C)
    ad = params["adapt"]
    dec = params["dec"]
    spatial_list = []
    for b in range(B):
        kp_idx = _fps(xyz_b[b], NKP)
        kp = xyz_b[b][kp_idx]
        idx_d, empty_d = _ball_query(kp, xyz_b[b], DEFORM_R, NS)
        rel = xyz_b[b][idx_d] - kp[:, None, :]
        gin = jnp.concatenate([rel, feat_b[b][idx_d]], -1)
        h = jax.nn.relu(gin @ ad["Wd1"] + ad["bd1"])
        w = jax.nn.softmax((h @ ad["Wd2"] + ad["bd2"])[..., 0], axis=-1)
        offset = jnp.sum(w[..., None] * rel, axis=1)
        def_xyz = kp + jnp.where(empty_d[:, None], 0.0, offset)
        def_xyz = jnp.clip(def_xyz, pcr_lo, pcr_hi)
        idx_p, empty_p = _ball_query(def_xyz, xyz_b[b], POOL_R, NS)
        local = _grouped_pointnet(xyz_b[b], feat_b[b], def_xyz, idx_p, empty_p,
                                  ad["Wa1"], ad["ba1"], ad["Wa2"], ad["ba2"])
        ctx = _sa_block(_sa_block(_sa_block(local, params["sa1"]), params["sa2"]), params["sa3"])
        idx_dec, empty_dec = _ball_query(xyz_b[b], def_xyz, DEC_R, NS)
        vox = _grouped_pointnet(def_xyz, ctx, xyz_b[b], idx_dec, empty_dec,
                                dec["W1"], dec["b1"], dec["W2"], dec["b2"])
        cb = coords.reshape(B, NB, 4)[b]
        indices = cb[:, 1] + cb[:, 2] * NX + cb[:, 3]
        spatial = jnp.zeros((C, NZ * NX * NY), vox.dtype).at[:, indices].set(vox.T)
        spatial_list.append(spatial)
    spatial_all = jnp.concatenate(spatial_list, 0)
    vc0 = spatial_all.reshape(B, C * NZ, NY, NX)
    vc1 = jax.image.resize(vc0, (B, C * NZ, NY // 2, NX // 2), method='bilinear', antialias=False)
    return vc0, vc1


# speedup vs baseline: 1.2398x; 1.2398x over previous
import jax
import jax.numpy as jnp
import numpy as np
from jax.experimental import pallas as pl

B = 2
NB = 12000
C = 64
NKP = 2048
NS = 16
NZ, NY, NX = 5, 200, 176
VOXEL_SIZE = np.array([0.05, 0.05, 0.1], np.float32)
PC_RANGE = np.array([0.0, -40.0, -3.0, 70.4, 40.0, 1.0], np.float32)
DEFORM_R = 2.0
POOL_R = 1.2
DEC_R = 1.6

_FR, _FCV, _FC = 8, 1500, 1536
_BIG = 1 << 20


def _fps_kernel(xs_ref, ys_ref, zs_ref, out_ref):
    xs = xs_ref[0]
    ys = ys_ref[0]
    zs = zs_ref[0]
    row = jax.lax.broadcasted_iota(jnp.int32, (_FR, _FC), 0)
    col = jax.lax.broadcasted_iota(jnp.int32, (_FR, _FC), 1)
    valid = col < _FCV
    flat = jnp.where(valid, row * _FCV + col, _BIG)
    oidx = (jax.lax.broadcasted_iota(jnp.int32, (NKP // 128, 128), 0) * 128
            + jax.lax.broadcasted_iota(jnp.int32, (NKP // 128, 128), 1))
    dists0 = jnp.where(valid, jnp.float32(1e10), jnp.float32(-1.0))
    idxs0 = jnp.zeros((NKP // 128, 128), jnp.int32)

    def body(i, st):
        dists, idxs, last = st
        sel = flat == last
        lx = jnp.sum(jnp.where(sel, xs, 0.0))
        ly = jnp.sum(jnp.where(sel, ys, 0.0))
        lz = jnp.sum(jnp.where(sel, zs, 0.0))
        dx = xs - lx
        dy = ys - ly
        dz = zs - lz
        d = dx * dx + dy * dy + dz * dz
        dists = jnp.minimum(dists, d)
        m = jnp.max(dists)
        nxt = jnp.min(jnp.where(dists == m, flat, _BIG)).astype(jnp.int32)
        idxs = jnp.where(oidx == i, nxt, idxs)
        return dists, idxs, nxt

    _, idxs, _ = jax.lax.fori_loop(1, NKP, body, (dists0, idxs0, jnp.int32(0)))
    out_ref[0] = idxs


def _fps_pallas(xyz_b):
    comps = []
    for a in range(3):
        c = xyz_b[:, :, a].reshape(B, _FR, _FCV)
        c = jnp.pad(c, ((0, 0), (0, 0), (0, _FC - _FCV)))
        comps.append(c)
    out = pl.pallas_call(
        _fps_kernel,
        grid=(B,),
        in_specs=[pl.BlockSpec((1, _FR, _FC), lambda b: (b, 0, 0))] * 3,
        out_specs=pl.BlockSpec((1, NKP // 128, 128), lambda b: (b, 0, 0)),
        out_shape=jax.ShapeDtypeStruct((B, NKP // 128, 128), jnp.int32),
    )(*comps)
    return out.reshape(B, NKP)


_QB = 256


def _make_bq_kernel(r2, n_true, n_pad):
    def kern(q_ref, px_ref, py_ref, pz_ref, idx_ref, emp_ref):
        q = q_ref[0]
        qx = q[:, 0:1]
        qy = q[:, 1:2]
        qz = q[:, 2:3]
        px = px_ref[0]
        py = py_ref[0]
        pz = pz_ref[0]
        dx = qx - px
        dy = qy - py
        dz = qz - pz
        d2 = dx * dx + dy * dy + dz * dz
        pid = jax.lax.broadcasted_iota(jnp.int32, (_QB, n_pad), 1)
        within = (d2 < r2) & (pid < n_true)
        key0 = jnp.where(within, pid, n_true)
        kcol = jax.lax.broadcasted_iota(jnp.int32, (_QB, NS), 1)

        def body(k, st):
            key, acc = st
            cur = jnp.min(key, axis=1, keepdims=True)
            acc = jnp.where(kcol == k, cur, acc)
            key = jnp.where(key == cur, n_true, key)
            return key, acc

        _, acc = jax.lax.fori_loop(
            0, NS, body, (key0, jnp.zeros((_QB, NS), jnp.int32)))
        first = acc[:, 0:1]
        emp = first == n_true
        acc = jnp.where(acc == n_true, first, acc)
        acc = jnp.where(emp, 0, acc)
        idx_ref[0] = acc
        emp_ref[0] = emp.astype(jnp.int32)

    return kern


def _ball_query_pallas(new_xyz, xyz, radius):
    nb, Q, _ = new_xyz.shape
    n = xyz.shape[1]
    n_pad = ((n + 127) // 128) * 128
    q_pad = ((Q + _QB - 1) // _QB) * _QB
    qarr = new_xyz
    if q_pad != Q:
        qarr = jnp.pad(qarr, ((0, 0), (0, q_pad - Q), (0, 0)),
                       constant_values=1e9)
    comps = []
    for a in range(3):
        c = xyz[:, :, a].reshape(nb, 1, n)
        if n_pad != n:
            c = jnp.pad(c, ((0, 0), (0, 0), (0, n_pad - n)))
        comps.append(c)
    kern = _make_bq_kernel(np.float32(radius * radius), n, n_pad)
    idx, emp = pl.pallas_call(
        kern,
        grid=(nb, q_pad // _QB),
        in_specs=[pl.BlockSpec((1, _QB, 3), lambda b, q: (b, q, 0))]
        + [pl.BlockSpec((1, 1, n_pad), lambda b, q: (b, 0, 0))] * 3,
        out_specs=[
            pl.BlockSpec((1, _QB, NS), lambda b, q: (b, q, 0)),
            pl.BlockSpec((1, _QB, 1), lambda b, q: (b, q, 0)),
        ],
        out_shape=[
            jax.ShapeDtypeStruct((nb, q_pad, NS), jnp.int32),
            jax.ShapeDtypeStruct((nb, q_pad, 1), jnp.int32),
        ],
    )(qarr, *comps)
    return idx[:, :Q], emp[:, :Q, 0].astype(jnp.bool_)


def _fps_jax(xyz):
    n = xyz.shape[0]

    def body(i, st):
        idxs, dists, last = st
        d = jnp.sum((xyz - xyz[last]) ** 2, -1)
        dists = jnp.minimum(dists, d)
        nxt = jnp.argmax(dists).astype(jnp.int32)
        return idxs.at[i].set(nxt), dists, nxt

    idxs = jnp.zeros((NKP,), jnp.int32)
    dists = jnp.full((n,), 1e10, jnp.float32)
    idxs, _, _ = jax.lax.fori_loop(1, NKP, body, (idxs, dists, jnp.int32(0)))
    return idxs


def _bq_xla(new_xyz, xyz, radius, nsample):
    n = xyz.shape[0]
    d2 = jnp.sum((new_xyz[:, None, :] - xyz[None, :, :]) ** 2, -1)
    within = d2 < radius * radius
    order = jnp.where(within, jnp.arange(n, dtype=jnp.int32)[None, :], n)
    neg_vals, _ = jax.lax.top_k(-order, nsample)
    idx = -neg_vals
    empty = ~jnp.any(within, axis=1)
    first = idx[:, :1]
    idx = jnp.where(idx == n, first, idx)
    idx = jnp.where(empty[:, None], 0, idx)
    return idx, empty


def _bq_xla_batched(new_xyz, xyz, radius):
    idx, emp = jax.vmap(lambda q, p: _bq_xla(q, p, radius, NS))(new_xyz, xyz)
    return idx, emp


def _grouped_pointnet(xyz_src, feat_src, centers, idx, empty, W1, b1, W2, b2):
    rel = xyz_src[idx] - centers[:, None, :]
    gf = jnp.concatenate([rel, feat_src[idx]], -1)
    h = jax.nn.relu(gf @ W1 + b1)
    h = jax.nn.relu(h @ W2 + b2)
    out = jnp.max(h, axis=1)
    return jnp.where(empty[:, None], 0.0, out)


def _sa_block(feats, p):
    t = feats @ p["Wt"]
    q = feats @ p["Wp"]
    g = feats @ p["Wg"]
    attn = jax.nn.softmax(t @ q.T, axis=-1)
    out = attn @ g
    og = out.reshape(-1, 4, C // 4)
    z = jnp.einsum('ngi,gij->ngj', og, p["Wz"]).reshape(-1, C)
    mu = z.mean()
    var = z.var()
    zn = (z - mu) / jnp.sqrt(var + 1e-5) * p["gamma"] + p["beta"]
    return zn + feats


def kernel(voxel_features, params, voxel_coords):
    coords = voxel_coords
    vs = jnp.asarray(VOXEL_SIZE) * 8.0
    pcr_lo = jnp.asarray(PC_RANGE[:3])
    pcr_hi = jnp.asarray(PC_RANGE[3:])
    xyz = (coords[:, jnp.array([3, 2, 1])].astype(jnp.float32) + 0.5) * vs + pcr_lo
    xyz_b = xyz.reshape(B, NB, 3)
    feat_b = voxel_features.reshape(B, NB, C)
    ad = params["adapt"]
    dec = params["dec"]

    kp_idx_all = jax.vmap(_fps_jax)(xyz_b)
    kp_all = jnp.take_along_axis(xyz_b, kp_idx_all[..., None], axis=1)
    idx_d_all, empty_d_all = _ball_query_pallas(kp_all, xyz_b, DEFORM_R)

    spatial_list = []
    def_list = []
    ctx_list = []
    for b in range(B):
        kp = kp_all[b]
        idx_d, empty_d = idx_d_all[b], empty_d_all[b]
        rel = xyz_b[b][idx_d] - kp[:, None, :]
        gin = jnp.concatenate([rel, feat_b[b][idx_d]], -1)
        h = jax.nn.relu(gin @ ad["Wd1"] + ad["bd1"])
        w = jax.nn.softmax((h @ ad["Wd2"] + ad["bd2"])[..., 0], axis=-1)
        offset = jnp.sum(w[..., None] * rel, axis=1)
        def_xyz = kp + jnp.where(empty_d[:, None], 0.0, offset)
        def_xyz = jnp.clip(def_xyz, pcr_lo, pcr_hi)
        def_list.append(def_xyz)

    def_all = jnp.stack(def_list, 0)
    idx_p_all, empty_p_all = _ball_query_pallas(def_all, xyz_b, POOL_R)
    for b in range(B):
        local = _grouped_pointnet(xyz_b[b], feat_b[b], def_all[b],
                                  idx_p_all[b], empty_p_all[b],
                                  ad["Wa1"], ad["ba1"], ad["Wa2"], ad["ba2"])
        ctx = _sa_block(_sa_block(_sa_block(local, params["sa1"]),
                                  params["sa2"]), params["sa3"])
        ctx_list.append(ctx)

    idx_dec_all, empty_dec_all = _ball_query_pallas(xyz_b, def_all, DEC_R)
    for b in range(B):
        vox = _grouped_pointnet(def_all[b], ctx_list[b], xyz_b[b],
                                idx_dec_all[b], empty_dec_all[b],
                                dec["W1"], dec["b1"], dec["W2"], dec["b2"])
        cb = coords.reshape(B, NB, 4)[b]
        indices = cb[:, 1] + cb[:, 2] * NX + cb[:, 3]
        spatial = jnp.zeros((C, NZ * NX * NY), vox.dtype).at[:, indices].set(vox.T)
        spatial_list.append(spatial)

    spatial_all = jnp.concatenate(spatial_list, 0)
    vc0 = spatial_all.reshape(B, C * NZ, NY, NX)
    vc1 = jax.image.resize(vc0, (B, C * NZ, NY // 2, NX // 2),
                           method='bilinear', antialias=False)
    return vc0, vc1
